# baseline (speedup 1.0000x reference)
import functools

import numpy as np
import jax
import jax.numpy as jnp
from jax import lax
from jax.experimental import pallas as pl
from jax.experimental.pallas import tpu as pltpu
from jax.experimental.pallas import tpu_sc as plsc

N = 10000
E = 320000
D_IN = 128
H1, C1 = 8, 16
C2 = 40

NC, NS = 2, 16
NW = NC * NS
B = 80
NB1 = E // (NS * B)
NB2 = E // (NW * B)
G = B // 16

ROWW1 = 80
ROWW2 = 48
ADW = 8
RCH = 80
TCH = 8


def _iota16():
    return lax.iota(jnp.int32, 16)


def _c16(v):
    return jnp.full((16,), v, dtype=jnp.int32)


def _make_sc_pass(roww, msgw, heads, ch, acol, td, nb):
    mesh = plsc.VectorSubcoreMesh(core_axis_name="c", subcore_axis_name="s",
                                  num_cores=NC, num_subcores=NS)
    f32 = jnp.float32
    out_type = (
        jax.ShapeDtypeStruct((N, msgw), f32),
        jax.ShapeDtypeStruct((N, msgw), f32),
        jax.ShapeDtypeStruct((N, ADW), f32),
        jax.ShapeDtypeStruct((N, ADW), f32),
    )
    scratch = [
        pltpu.VMEM((nb, B), jnp.int32),
        pltpu.VMEM((nb, B), jnp.int32),
        pltpu.VMEM((3, B, roww), f32),
        pltpu.VMEM((3, B, ADW), f32),
        pltpu.VMEM((3, B, ADW), f32),
        pltpu.VMEM((3, B, msgw), f32),
        pltpu.VMEM((RCH, msgw), f32),
        pltpu.VMEM((RCH, ADW), f32),
        pltpu.VMEM_SHARED((N, msgw), f32),
        pltpu.VMEM_SHARED((N, ADW), f32),
    ] + [pltpu.SemaphoreType.DMA] * 12

    @functools.partial(
        pl.kernel, out_type=out_type, mesh=mesh, scratch_types=scratch,
        compiler_params=pltpu.CompilerParams(use_tc_tiling_on_sc=False,
                                             needs_layout_passes=False))
    def sc_pass(srcI, dstI, Hs, Ad, num0, num1, den0, den1,
                src_v, dst_v, rows2, ad2, ea2, msg2, zb, zbd,
                acc_num, acc_den, *sems):
        sgH, sgA, ssE, ssM = (sems[0:3], sems[3:6], sems[6:9],
                              sems[9:12])
        rows = tuple(rows2.at[b] for b in range(3))
        adv = tuple(ad2.at[b] for b in range(3))
        eav = tuple(ea2.at[b] for b in range(3))
        msgv = tuple(msg2.at[b] for b in range(3))
        cid = lax.axis_index("c")
        sid = lax.axis_index("s")
        wid = sid if td == 2 else cid * NS + sid
        tix = cid if td == 2 else 0
        z16 = jnp.zeros((16,), f32)
        iota = _iota16()
        r2 = iota // ADW
        c8 = iota % ADW

        for r in range(RCH):
            for k in range(msgw // 16):
                zb[r, pl.ds(k * 16, 16)] = z16
        for r in range(0, RCH, 2):
            plsc.store_scatter(zbd, [r + r2, c8], z16)
        for b in range(3):
            for r in range(0, B, 2):
                plsc.store_scatter(eav[b], [r + r2, c8], z16)
            for r in range(B):
                for k in range(msgw // 16):
                    msg2[b, r, pl.ds(k * 16, 16)] = z16

        pltpu.async_copy(srcI.at[wid], src_v, sems[0])
        pltpu.async_copy(dstI.at[wid], dst_v, sems[1])
        for t in range(TCH):
            base = (sid * TCH + t) * RCH

            @pl.when(base < N)
            def _():
                pltpu.async_copy(zb, acc_num.at[pl.ds(base, RCH)], sems[2])
                pltpu.async_copy(zbd, acc_den.at[pl.ds(base, RCH)], sems[3])
        for t in range(TCH):
            base = (sid * TCH + t) * RCH

            @pl.when(base < N)
            def _():
                pltpu.make_async_copy(zb, acc_num.at[pl.ds(base, RCH)],
                                      sems[2]).wait()
                pltpu.make_async_copy(zbd, acc_den.at[pl.ds(base, RCH)],
                                      sems[3]).wait()
        pltpu.make_async_copy(srcI.at[wid], src_v, sems[0]).wait()
        pltpu.make_async_copy(dstI.at[wid], dst_v, sems[1]).wait()
        plsc.subcore_barrier()

        for b in range(3):
            pltpu.async_copy(eav[b], acc_den.at[dst_v.at[0]], ssE[b],
                             add=True)
            pltpu.async_copy(msgv[b], acc_num.at[dst_v.at[0]], ssM[b],
                             add=True)
            pltpu.async_copy(Hs.at[tix].at[src_v.at[b]], rows[b], sgH[b])
            pltpu.async_copy(Ad.at[tix].at[dst_v.at[b]], adv[b], sgA[b])

        def do_block(j, b):
            pltpu.make_async_copy(eav[b], acc_den.at[dst_v.at[0]],
                                  ssE[b]).wait()
            pltpu.make_async_copy(msgv[b], acc_num.at[dst_v.at[0]],
                                  ssM[b]).wait()
            pltpu.make_async_copy(Hs.at[tix].at[src_v.at[0]], rows[b],
                                  sgH[b]).wait()
            pltpu.make_async_copy(Ad.at[tix].at[dst_v.at[0]], adv[b],
                                  sgA[b]).wait()
            nk = (ch + 15) // 16
            for g in range(G):
                e16 = iota + (g * 16)
                eas = []
                for h in range(heads):
                    a_s = plsc.load_gather(rows[b], [e16, _c16(acol + h)])
                    a_d = plsc.load_gather(adv[b], [e16, _c16(h)])
                    alpha = a_s + a_d
                    alpha = jnp.where(alpha > 0.0, alpha, alpha * 0.2)
                    ea = jnp.exp(alpha)
                    plsc.store_scatter(eav[b], [e16, _c16(h)], ea)
                    eas.append(ea)
                for e in range(16):
                    r = g * 16 + e
                    for h in range(heads):
                        eab = eas[h].at[_c16(e)].get(
                            mode="promise_in_bounds")
                        for k in range(nk):
                            col = h * ch + k * 16
                            hv = rows2[b, r, pl.ds(col, 16)]
                            msg2[b, r, pl.ds(col, 16)] = hv * eab
            pltpu.async_copy(eav[b], acc_den.at[dst_v.at[j]], ssE[b],
                             add=True)
            pltpu.async_copy(msgv[b], acc_num.at[dst_v.at[j]], ssM[b],
                             add=True)

            @pl.when(j + 3 < nb)
            def _():
                pltpu.async_copy(Hs.at[tix].at[src_v.at[j + 3]], rows[b],
                                 sgH[b])
                pltpu.async_copy(Ad.at[tix].at[dst_v.at[j + 3]], adv[b],
                                 sgA[b])

        @pl.loop(0, 3 * (nb // 3), step=3)
        def _blk(j0):
            for b in range(3):
                do_block(j0 + b, b)

        for j in range(3 * (nb // 3), nb):
            do_block(j, j % 3)

        for b in range(3):
            pltpu.make_async_copy(eav[b], acc_den.at[dst_v.at[0]],
                                  ssE[b]).wait()
            pltpu.make_async_copy(msgv[b], acc_num.at[dst_v.at[0]],
                                  ssM[b]).wait()

        plsc.subcore_barrier()

        for t in range(TCH):
            base = (sid * TCH + t) * RCH
            s = pl.ds(base, RCH)

            @pl.when((cid == 0) & (base < N))
            def _():
                pltpu.sync_copy(acc_num.at[s], num0.at[s])
                pltpu.sync_copy(acc_den.at[s], den0.at[s])

            @pl.when((cid == 1) & (base < N))
            def _():
                pltpu.sync_copy(acc_num.at[s], num1.at[s])
                pltpu.sync_copy(acc_den.at[s], den1.at[s])

    return sc_pass


_sc_pass1 = _make_sc_pass(ROWW1, 64, H1 // 2, C1, 64, 2, NB1)
_sc_pass2 = _make_sc_pass(ROWW2, ROWW2, 1, C2, 40, 1, NB2)


_RB = 1000


def _tc1_body(x_ref, w_ref, ms_ref, md_ref, hs_ref, ad_ref):
    h = jnp.dot(x_ref[...], w_ref[...], preferred_element_type=jnp.float32)
    a_s = jnp.dot(h, ms_ref[...], preferred_element_type=jnp.float32)
    a_d = jnp.dot(h, md_ref[...], preferred_element_type=jnp.float32)
    z12 = jnp.zeros((h.shape[0], 12), jnp.float32)
    z4 = jnp.zeros((h.shape[0], 4), jnp.float32)
    hs_ref[0] = jnp.concatenate([h[:, :64], a_s[:, :4], z12], axis=1)
    hs_ref[1] = jnp.concatenate([h[:, 64:], a_s[:, 4:], z12], axis=1)
    ad_ref[0] = jnp.concatenate([a_d[:, :4], z4], axis=1)
    ad_ref[1] = jnp.concatenate([a_d[:, 4:], z4], axis=1)


def _tc1(x, W1, Msrc, Mdst):
    return pl.pallas_call(
        _tc1_body,
        grid=(N // _RB,),
        in_specs=[
            pl.BlockSpec((_RB, D_IN), lambda i: (i, 0)),
            pl.BlockSpec((D_IN, D_IN), lambda i: (0, 0)),
            pl.BlockSpec((D_IN, H1), lambda i: (0, 0)),
            pl.BlockSpec((D_IN, H1), lambda i: (0, 0)),
        ],
        out_specs=[
            pl.BlockSpec((2, _RB, ROWW1), lambda i: (0, i, 0)),
            pl.BlockSpec((2, _RB, ADW), lambda i: (0, i, 0)),
        ],
        out_shape=[
            jax.ShapeDtypeStruct((2, N, ROWW1), jnp.float32),
            jax.ShapeDtypeStruct((2, N, ADW), jnp.float32),
        ],
    )(x, W1, Msrc, Mdst)


def _tc2_body(n0_ref, n1_ref, d0_ref, d1_ref, ex_ref, b1_ref, w2_ref,
              as2_ref, ad2_ref, hs_ref, ad_ref):
    num = jnp.concatenate([n0_ref[...], n1_ref[...]], axis=1)
    den = jnp.concatenate([d0_ref[:, :4], d1_ref[:, :4]], axis=1)
    den128 = jnp.dot(den, ex_ref[...], preferred_element_type=jnp.float32)
    out1 = num / (den128 + 1e-16) + b1_ref[...]
    y = jnp.where(out1 > 0.0, out1, jnp.exp(out1) - 1.0)
    h2 = jnp.dot(y, w2_ref[...], preferred_element_type=jnp.float32)
    a_s = jnp.dot(h2, as2_ref[...], preferred_element_type=jnp.float32)
    a_d = jnp.dot(h2, ad2_ref[...], preferred_element_type=jnp.float32)
    hs_ref[...] = jnp.concatenate([h2, a_s], axis=1)
    ad_ref[...] = a_d


def _tc2(n0, n1, d0, d1, Ex1, b1r, W2, as2m, ad2m):
    return pl.pallas_call(
        _tc2_body,
        grid=(N // _RB,),
        in_specs=[
            pl.BlockSpec((_RB, 64), lambda i: (i, 0)),
            pl.BlockSpec((_RB, 64), lambda i: (i, 0)),
            pl.BlockSpec((_RB, ADW), lambda i: (i, 0)),
            pl.BlockSpec((_RB, ADW), lambda i: (i, 0)),
            pl.BlockSpec((H1, 128), lambda i: (0, 0)),
            pl.BlockSpec((1, 128), lambda i: (0, 0)),
            pl.BlockSpec((128, C2), lambda i: (0, 0)),
            pl.BlockSpec((C2, 8), lambda i: (0, 0)),
            pl.BlockSpec((C2, ADW), lambda i: (0, 0)),
        ],
        out_specs=[
            pl.BlockSpec((_RB, ROWW2), lambda i: (i, 0)),
            pl.BlockSpec((_RB, ADW), lambda i: (i, 0)),
        ],
        out_shape=[
            jax.ShapeDtypeStruct((N, ROWW2), jnp.float32),
            jax.ShapeDtypeStruct((N, ADW), jnp.float32),
        ],
    )(n0, n1, d0, d1, Ex1, b1r, W2, as2m, ad2m)


def _tc3_body(n0_ref, n1_ref, d0_ref, d1_ref, ex_ref, b2_ref, o_ref):
    den = d0_ref[...] + d1_ref[...]
    den40 = jnp.dot(den, ex_ref[...], preferred_element_type=jnp.float32)
    o_ref[...] = ((n0_ref[...] + n1_ref[...])[:, :C2] / (den40 + 1e-16)
                  + b2_ref[...])


def _tc3(n0, n1, d0, d1, Ex2, b2r):
    return pl.pallas_call(
        _tc3_body,
        grid=(N // _RB,),
        in_specs=[
            pl.BlockSpec((_RB, ROWW2), lambda i: (i, 0)),
            pl.BlockSpec((_RB, ROWW2), lambda i: (i, 0)),
            pl.BlockSpec((_RB, ADW), lambda i: (i, 0)),
            pl.BlockSpec((_RB, ADW), lambda i: (i, 0)),
            pl.BlockSpec((ADW, C2), lambda i: (0, 0)),
            pl.BlockSpec((1, C2), lambda i: (0, 0)),
        ],
        out_specs=pl.BlockSpec((_RB, C2), lambda i: (i, 0)),
        out_shape=jax.ShapeDtypeStruct((N, C2), jnp.float32),
    )(n0, n1, d0, d1, Ex2, b2r)


_EX1 = np.zeros((H1, 128), np.float32)
for _h in range(H1):
    _EX1[_h, _h * C1:(_h + 1) * C1] = 1.0
_EX2 = np.zeros((ADW, C2), np.float32)
_EX2[0, :] = 1.0


def kernel(x, edge_index, W1, att_src1, att_dst1, b1, W2, att_src2,
           att_dst2, b2):
    srcI1 = edge_index[0].reshape(NS, NB1, B)
    dstI1 = edge_index[1].reshape(NS, NB1, B)
    srcI2 = edge_index[0].reshape(NW, NB2, B)
    dstI2 = edge_index[1].reshape(NW, NB2, B)

    a1s = att_src1.reshape(H1, C1)
    a1d = att_dst1.reshape(H1, C1)
    eye8 = jnp.eye(H1, dtype=jnp.float32)
    Msrc = (a1s[:, :, None] * eye8[:, None, :]).reshape(H1 * C1, H1)
    Mdst = (a1d[:, :, None] * eye8[:, None, :]).reshape(H1 * C1, H1)

    Hs1, Ad1 = _tc1(x, W1, Msrc, Mdst)
    n10, n11, d10, d11 = _sc_pass1(srcI1, dstI1, Hs1, Ad1)

    as2m = jnp.pad(att_src2.reshape(C2, 1), ((0, 0), (0, 7)))
    ad2m = jnp.pad(att_dst2.reshape(C2, 1), ((0, 0), (0, ADW - 1)))
    Hs2, Ad2 = _tc2(n10, n11, d10, d11, jnp.asarray(_EX1),
                    b1.reshape(1, 128), W2, as2m, ad2m)
    n20, n21, d20, d21 = _sc_pass2(srcI2, dstI2, Hs2[None], Ad2[None])

    return _tc3(n20, n21, d20, d21, jnp.asarray(_EX2), b2.reshape(1, C2))


# speedup vs baseline: 1.0469x; 1.0469x over previous
import functools

import numpy as np
import jax
import jax.numpy as jnp
from jax import lax
from jax.experimental import pallas as pl
from jax.experimental.pallas import tpu as pltpu
from jax.experimental.pallas import tpu_sc as plsc

N = 10000
E = 320000
D_IN = 128
H1, C1 = 8, 16
C2 = 40

NC, NS = 2, 16
NW = NC * NS
B = 80
NB1 = E // (NS * B)
NB2 = E // (NW * B)
G = B // 16

ROWW1 = 72
ROWW2 = 48
ADW = 8
RCH = 80
TCH = 8


def _iota16():
    return lax.iota(jnp.int32, 16)


def _c16(v):
    return jnp.full((16,), v, dtype=jnp.int32)


def _make_sc_pass(roww, msgw, heads, ch, acol, td, nb):
    mesh = plsc.VectorSubcoreMesh(core_axis_name="c", subcore_axis_name="s",
                                  num_cores=NC, num_subcores=NS)
    f32 = jnp.float32
    out_type = (
        jax.ShapeDtypeStruct((N, msgw), f32),
        jax.ShapeDtypeStruct((N, msgw), f32),
        jax.ShapeDtypeStruct((N, ADW), f32),
        jax.ShapeDtypeStruct((N, ADW), f32),
    )
    scratch = [
        pltpu.VMEM((nb, B), jnp.int32),
        pltpu.VMEM((nb, B), jnp.int32),
        pltpu.VMEM((3, B, roww), f32),
        pltpu.VMEM((3, B, ADW), f32),
        pltpu.VMEM((3, B, ADW), f32),
        pltpu.VMEM((3, B, msgw), f32),
        pltpu.VMEM((RCH, msgw), f32),
        pltpu.VMEM((RCH, ADW), f32),
        pltpu.VMEM_SHARED((N, msgw), f32),
        pltpu.VMEM_SHARED((N, ADW), f32),
    ] + [pltpu.SemaphoreType.DMA] * 12

    @functools.partial(
        pl.kernel, out_type=out_type, mesh=mesh, scratch_types=scratch,
        compiler_params=pltpu.CompilerParams(use_tc_tiling_on_sc=False,
                                             needs_layout_passes=False))
    def sc_pass(srcI, dstI, Hs, Ad, num0, num1, den0, den1,
                src_v, dst_v, rows2, ad2, ea2, msg2, zb, zbd,
                acc_num, acc_den, *sems):
        sgH, sgA, ssE, ssM = (sems[0:3], sems[3:6], sems[6:9],
                              sems[9:12])
        rows = tuple(rows2.at[b] for b in range(3))
        adv = tuple(ad2.at[b] for b in range(3))
        eav = tuple(ea2.at[b] for b in range(3))
        msgv = tuple(msg2.at[b] for b in range(3))
        cid = lax.axis_index("c")
        sid = lax.axis_index("s")
        wid = sid if td == 2 else cid * NS + sid
        tix = cid if td == 2 else 0
        z16 = jnp.zeros((16,), f32)
        iota = _iota16()
        r2 = iota // ADW
        c8 = iota % ADW

        for r in range(RCH):
            for k in range(msgw // 16):
                zb[r, pl.ds(k * 16, 16)] = z16
        for r in range(0, RCH, 2):
            plsc.store_scatter(zbd, [r + r2, c8], z16)
        for b in range(3):
            for r in range(0, B, 2):
                plsc.store_scatter(eav[b], [r + r2, c8], z16)
            for r in range(B):
                for k in range(msgw // 16):
                    msg2[b, r, pl.ds(k * 16, 16)] = z16

        pltpu.async_copy(srcI.at[wid], src_v, sems[0])
        pltpu.async_copy(dstI.at[wid], dst_v, sems[1])
        for t in range(TCH):
            base = (sid * TCH + t) * RCH

            @pl.when(base < N)
            def _():
                pltpu.async_copy(zb, acc_num.at[pl.ds(base, RCH)], sems[2])
                pltpu.async_copy(zbd, acc_den.at[pl.ds(base, RCH)], sems[3])
        for t in range(TCH):
            base = (sid * TCH + t) * RCH

            @pl.when(base < N)
            def _():
                pltpu.make_async_copy(zb, acc_num.at[pl.ds(base, RCH)],
                                      sems[2]).wait()
                pltpu.make_async_copy(zbd, acc_den.at[pl.ds(base, RCH)],
                                      sems[3]).wait()
        pltpu.make_async_copy(srcI.at[wid], src_v, sems[0]).wait()
        pltpu.make_async_copy(dstI.at[wid], dst_v, sems[1]).wait()
        plsc.subcore_barrier()

        for b in range(3):
            pltpu.async_copy(eav[b], acc_den.at[dst_v.at[0]], ssE[b],
                             add=True)
            pltpu.async_copy(msgv[b], acc_num.at[dst_v.at[0]], ssM[b],
                             add=True)
            pltpu.async_copy(Hs.at[tix].at[src_v.at[b]], rows[b], sgH[b])
            pltpu.async_copy(Ad.at[tix].at[dst_v.at[b]], adv[b], sgA[b])

        def do_block(j, b):
            pltpu.make_async_copy(eav[b], acc_den.at[dst_v.at[0]],
                                  ssE[b]).wait()
            pltpu.make_async_copy(msgv[b], acc_num.at[dst_v.at[0]],
                                  ssM[b]).wait()
            pltpu.make_async_copy(Hs.at[tix].at[src_v.at[0]], rows[b],
                                  sgH[b]).wait()
            pltpu.make_async_copy(Ad.at[tix].at[dst_v.at[0]], adv[b],
                                  sgA[b]).wait()
            nk = (ch + 15) // 16
            for g in range(G):
                e16 = iota + (g * 16)
                eas = []
                for h in range(heads):
                    a_s = plsc.load_gather(rows[b], [e16, _c16(acol + h)])
                    a_d = plsc.load_gather(adv[b], [e16, _c16(h)])
                    alpha = a_s + a_d
                    alpha = jnp.where(alpha > 0.0, alpha, alpha * 0.2)
                    ea = jnp.exp(alpha)
                    plsc.store_scatter(eav[b], [e16, _c16(h)], ea)
                    eas.append(ea)
                for e in range(16):
                    r = g * 16 + e
                    for h in range(heads):
                        eab = eas[h].at[_c16(e)].get(
                            mode="promise_in_bounds")
                        for k in range(nk):
                            col = h * ch + k * 16
                            hv = rows2[b, r, pl.ds(col, 16)]
                            msg2[b, r, pl.ds(col, 16)] = hv * eab
            pltpu.async_copy(eav[b], acc_den.at[dst_v.at[j]], ssE[b],
                             add=True)
            pltpu.async_copy(msgv[b], acc_num.at[dst_v.at[j]], ssM[b],
                             add=True)

            @pl.when(j + 3 < nb)
            def _():
                pltpu.async_copy(Hs.at[tix].at[src_v.at[j + 3]], rows[b],
                                 sgH[b])
                pltpu.async_copy(Ad.at[tix].at[dst_v.at[j + 3]], adv[b],
                                 sgA[b])

        @pl.loop(0, 3 * (nb // 3), step=3)
        def _blk(j0):
            for b in range(3):
                do_block(j0 + b, b)

        for j in range(3 * (nb // 3), nb):
            do_block(j, j % 3)

        for b in range(3):
            pltpu.make_async_copy(eav[b], acc_den.at[dst_v.at[0]],
                                  ssE[b]).wait()
            pltpu.make_async_copy(msgv[b], acc_num.at[dst_v.at[0]],
                                  ssM[b]).wait()

        plsc.subcore_barrier()

        for t in range(TCH):
            base = (sid * TCH + t) * RCH
            s = pl.ds(base, RCH)

            @pl.when((cid == 0) & (base < N))
            def _():
                pltpu.async_copy(acc_num.at[s], num0.at[s], sems[0])
                pltpu.async_copy(acc_den.at[s], den0.at[s], sems[1])

            @pl.when((cid == 1) & (base < N))
            def _():
                pltpu.async_copy(acc_num.at[s], num1.at[s], sems[0])
                pltpu.async_copy(acc_den.at[s], den1.at[s], sems[1])
        for t in range(TCH):
            base = (sid * TCH + t) * RCH
            s = pl.ds(base, RCH)

            @pl.when((cid == 0) & (base < N))
            def _():
                pltpu.make_async_copy(acc_num.at[s], num0.at[s],
                                      sems[0]).wait()
                pltpu.make_async_copy(acc_den.at[s], den0.at[s],
                                      sems[1]).wait()

            @pl.when((cid == 1) & (base < N))
            def _():
                pltpu.make_async_copy(acc_num.at[s], num1.at[s],
                                      sems[0]).wait()
                pltpu.make_async_copy(acc_den.at[s], den1.at[s],
                                      sems[1]).wait()

    return sc_pass


_sc_pass1 = _make_sc_pass(ROWW1, 64, H1 // 2, C1, 64, 2, NB1)
_sc_pass2 = _make_sc_pass(ROWW2, ROWW2, 1, C2, 40, 1, NB2)


_RB = 1000


def _tc1_body(x_ref, w_ref, ms_ref, md_ref, hs_ref, ad_ref):
    h = jnp.dot(x_ref[...], w_ref[...], preferred_element_type=jnp.float32)
    a_s = jnp.dot(h, ms_ref[...], preferred_element_type=jnp.float32)
    a_d = jnp.dot(h, md_ref[...], preferred_element_type=jnp.float32)
    z12 = jnp.zeros((h.shape[0], 4), jnp.float32)
    z4 = jnp.zeros((h.shape[0], 4), jnp.float32)
    hs_ref[0] = jnp.concatenate([h[:, :64], a_s[:, :4], z12], axis=1)
    hs_ref[1] = jnp.concatenate([h[:, 64:], a_s[:, 4:], z12], axis=1)
    ad_ref[0] = jnp.concatenate([a_d[:, :4], z4], axis=1)
    ad_ref[1] = jnp.concatenate([a_d[:, 4:], z4], axis=1)


def _tc1(x, W1, Msrc, Mdst):
    return pl.pallas_call(
        _tc1_body,
        grid=(N // _RB,),
        in_specs=[
            pl.BlockSpec((_RB, D_IN), lambda i: (i, 0)),
            pl.BlockSpec((D_IN, D_IN), lambda i: (0, 0)),
            pl.BlockSpec((D_IN, H1), lambda i: (0, 0)),
            pl.BlockSpec((D_IN, H1), lambda i: (0, 0)),
        ],
        out_specs=[
            pl.BlockSpec((2, _RB, ROWW1), lambda i: (0, i, 0)),
            pl.BlockSpec((2, _RB, ADW), lambda i: (0, i, 0)),
        ],
        out_shape=[
            jax.ShapeDtypeStruct((2, N, ROWW1), jnp.float32),
            jax.ShapeDtypeStruct((2, N, ADW), jnp.float32),
        ],
    )(x, W1, Msrc, Mdst)


def _tc2_body(n0_ref, n1_ref, d0_ref, d1_ref, ex_ref, b1_ref, w2_ref,
              as2_ref, ad2_ref, hs_ref, ad_ref):
    num = jnp.concatenate([n0_ref[...], n1_ref[...]], axis=1)
    den = jnp.concatenate([d0_ref[:, :4], d1_ref[:, :4]], axis=1)
    den128 = jnp.dot(den, ex_ref[...], preferred_element_type=jnp.float32)
    out1 = num / (den128 + 1e-16) + b1_ref[...]
    y = jnp.where(out1 > 0.0, out1, jnp.exp(out1) - 1.0)
    h2 = jnp.dot(y, w2_ref[...], preferred_element_type=jnp.float32)
    a_s = jnp.dot(h2, as2_ref[...], preferred_element_type=jnp.float32)
    a_d = jnp.dot(h2, ad2_ref[...], preferred_element_type=jnp.float32)
    hs_ref[...] = jnp.concatenate([h2, a_s], axis=1)
    ad_ref[...] = a_d


def _tc2(n0, n1, d0, d1, Ex1, b1r, W2, as2m, ad2m):
    return pl.pallas_call(
        _tc2_body,
        grid=(N // _RB,),
        in_specs=[
            pl.BlockSpec((_RB, 64), lambda i: (i, 0)),
            pl.BlockSpec((_RB, 64), lambda i: (i, 0)),
            pl.BlockSpec((_RB, ADW), lambda i: (i, 0)),
            pl.BlockSpec((_RB, ADW), lambda i: (i, 0)),
            pl.BlockSpec((H1, 128), lambda i: (0, 0)),
            pl.BlockSpec((1, 128), lambda i: (0, 0)),
            pl.BlockSpec((128, C2), lambda i: (0, 0)),
            pl.BlockSpec((C2, 8), lambda i: (0, 0)),
            pl.BlockSpec((C2, ADW), lambda i: (0, 0)),
        ],
        out_specs=[
            pl.BlockSpec((_RB, ROWW2), lambda i: (i, 0)),
            pl.BlockSpec((_RB, ADW), lambda i: (i, 0)),
        ],
        out_shape=[
            jax.ShapeDtypeStruct((N, ROWW2), jnp.float32),
            jax.ShapeDtypeStruct((N, ADW), jnp.float32),
        ],
    )(n0, n1, d0, d1, Ex1, b1r, W2, as2m, ad2m)


def _tc3_body(n0_ref, n1_ref, d0_ref, d1_ref, ex_ref, b2_ref, o_ref):
    den = d0_ref[...] + d1_ref[...]
    den40 = jnp.dot(den, ex_ref[...], preferred_element_type=jnp.float32)
    o_ref[...] = ((n0_ref[...] + n1_ref[...])[:, :C2] / (den40 + 1e-16)
                  + b2_ref[...])


def _tc3(n0, n1, d0, d1, Ex2, b2r):
    return pl.pallas_call(
        _tc3_body,
        grid=(N // _RB,),
        in_specs=[
            pl.BlockSpec((_RB, ROWW2), lambda i: (i, 0)),
            pl.BlockSpec((_RB, ROWW2), lambda i: (i, 0)),
            pl.BlockSpec((_RB, ADW), lambda i: (i, 0)),
            pl.BlockSpec((_RB, ADW), lambda i: (i, 0)),
            pl.BlockSpec((ADW, C2), lambda i: (0, 0)),
            pl.BlockSpec((1, C2), lambda i: (0, 0)),
        ],
        out_specs=pl.BlockSpec((_RB, C2), lambda i: (i, 0)),
        out_shape=jax.ShapeDtypeStruct((N, C2), jnp.float32),
    )(n0, n1, d0, d1, Ex2, b2r)


_EX1 = np.zeros((H1, 128), np.float32)
for _h in range(H1):
    _EX1[_h, _h * C1:(_h + 1) * C1] = 1.0
_EX2 = np.zeros((ADW, C2), np.float32)
_EX2[0, :] = 1.0


def kernel(x, edge_index, W1, att_src1, att_dst1, b1, W2, att_src2,
           att_dst2, b2):
    srcI1 = edge_index[0].reshape(NS, NB1, B)
    dstI1 = edge_index[1].reshape(NS, NB1, B)
    srcI2 = edge_index[0].reshape(NW, NB2, B)
    dstI2 = edge_index[1].reshape(NW, NB2, B)

    a1s = att_src1.reshape(H1, C1)
    a1d = att_dst1.reshape(H1, C1)
    eye8 = jnp.eye(H1, dtype=jnp.float32)
    Msrc = (a1s[:, :, None] * eye8[:, None, :]).reshape(H1 * C1, H1)
    Mdst = (a1d[:, :, None] * eye8[:, None, :]).reshape(H1 * C1, H1)

    Hs1, Ad1 = _tc1(x, W1, Msrc, Mdst)
    n10, n11, d10, d11 = _sc_pass1(srcI1, dstI1, Hs1, Ad1)

    as2m = jnp.pad(att_src2.reshape(C2, 1), ((0, 0), (0, 7)))
    ad2m = jnp.pad(att_dst2.reshape(C2, 1), ((0, 0), (0, ADW - 1)))
    Hs2, Ad2 = _tc2(n10, n11, d10, d11, jnp.asarray(_EX1),
                    b1.reshape(1, 128), W2, as2m, ad2m)
    n20, n21, d20, d21 = _sc_pass2(srcI2, dstI2, Hs2[None], Ad2[None])

    return _tc3(n20, n21, d20, d21, jnp.asarray(_EX2), b2.reshape(1, C2))


# speedup vs baseline: 1.2726x; 1.2156x over previous
import functools

import numpy as np
import jax
import jax.numpy as jnp
from jax import lax
from jax.experimental import pallas as pl
from jax.experimental.pallas import tpu as pltpu
from jax.experimental.pallas import tpu_sc as plsc

N = 10000
E = 320000
D_IN = 128
H1, C1 = 8, 16
C2 = 40

NC, NS = 2, 16
NW = NC * NS
B = 80
NB1 = E // (NS * B)
NB2 = E // (NW * B)
G = B // 16

ROWW1 = 72
ROWW2 = 48
ADW = 8
RCH = 80
TCH = 8


def _iota16():
    return lax.iota(jnp.int32, 16)


def _c16(v):
    return jnp.full((16,), v, dtype=jnp.int32)


def _make_sc_pass(roww, heads, ch, acol, eacol, accw, td, nb):
    mesh = plsc.VectorSubcoreMesh(core_axis_name="c", subcore_axis_name="s",
                                  num_cores=NC, num_subcores=NS)
    f32 = jnp.float32
    out_type = (
        jax.ShapeDtypeStruct((N, accw), f32),
        jax.ShapeDtypeStruct((N, accw), f32),
    )
    scratch = [
        pltpu.VMEM((nb, B), jnp.int32),
        pltpu.VMEM((nb, B), jnp.int32),
        pltpu.VMEM((3, B, roww), f32),
        pltpu.VMEM((3, B, ADW), f32),
        pltpu.VMEM((3, B, accw), f32),
        pltpu.VMEM((RCH, accw), f32),
        pltpu.VMEM_SHARED((N, accw), f32),
    ] + [pltpu.SemaphoreType.DMA] * 9

    nzk = (accw + 15) // 16

    @functools.partial(
        pl.kernel, out_type=out_type, mesh=mesh, scratch_types=scratch,
        compiler_params=pltpu.CompilerParams(use_tc_tiling_on_sc=False,
                                             needs_layout_passes=False))
    def sc_pass(srcI, dstI, Hs, Ad, num0, num1,
                src_v, dst_v, rows2, ad2, msg2, zb, acc, *sems):
        sgH, sgA, ssM = sems[0:3], sems[3:6], sems[6:9]
        rows = tuple(rows2.at[b] for b in range(3))
        adv = tuple(ad2.at[b] for b in range(3))
        msgv = tuple(msg2.at[b] for b in range(3))
        cid = lax.axis_index("c")
        sid = lax.axis_index("s")
        wid = sid if td == 2 else cid * NS + sid
        tix = cid if td == 2 else 0
        z16 = jnp.zeros((16,), f32)
        iota = _iota16()

        for r in range(RCH):
            for k in range(nzk):
                zb[r, pl.ds(min(k * 16, accw - 16), 16)] = z16
        for b in range(3):
            for r in range(B):
                for k in range(nzk):
                    msg2[b, r, pl.ds(min(k * 16, accw - 16), 16)] = z16

        pltpu.async_copy(srcI.at[wid], src_v, sems[0])
        pltpu.async_copy(dstI.at[wid], dst_v, sems[1])
        for t in range(TCH):
            base = (sid * TCH + t) * RCH

            @pl.when(base < N)
            def _():
                pltpu.async_copy(zb, acc.at[pl.ds(base, RCH)], sems[2])
        for t in range(TCH):
            base = (sid * TCH + t) * RCH

            @pl.when(base < N)
            def _():
                pltpu.make_async_copy(zb, acc.at[pl.ds(base, RCH)],
                                      sems[2]).wait()
        pltpu.make_async_copy(srcI.at[wid], src_v, sems[0]).wait()
        pltpu.make_async_copy(dstI.at[wid], dst_v, sems[1]).wait()
        plsc.subcore_barrier()

        for b in range(3):
            pltpu.async_copy(msgv[b], acc.at[dst_v.at[0]], ssM[b], add=True)
            pltpu.async_copy(Hs.at[tix].at[src_v.at[b]], rows[b], sgH[b])
            pltpu.async_copy(Ad.at[tix].at[dst_v.at[b]], adv[b], sgA[b])

        def do_block(j, b):
            pltpu.make_async_copy(msgv[b], acc.at[dst_v.at[0]],
                                  ssM[b]).wait()
            pltpu.make_async_copy(Hs.at[tix].at[src_v.at[0]], rows[b],
                                  sgH[b]).wait()
            pltpu.make_async_copy(Ad.at[tix].at[dst_v.at[0]], adv[b],
                                  sgA[b]).wait()
            nk = (ch + 15) // 16
            for g in range(G):
                e16 = iota + (g * 16)
                eas = []
                for h in range(heads):
                    a_s = plsc.load_gather(rows[b], [e16, _c16(acol + h)])
                    a_d = plsc.load_gather(adv[b], [e16, _c16(h)])
                    alpha = a_s + a_d
                    alpha = jnp.where(alpha > 0.0, alpha, alpha * 0.2)
                    eas.append(jnp.exp(alpha))
                for e in range(16):
                    r = g * 16 + e
                    for h in range(heads):
                        eab = eas[h].at[_c16(e)].get(
                            mode="promise_in_bounds")
                        for k in range(nk):
                            col = h * ch + k * 16
                            hv = rows2[b, r, pl.ds(col, 16)]
                            msg2[b, r, pl.ds(col, 16)] = hv * eab
                for h in range(heads):
                    plsc.store_scatter(msgv[b], [e16, _c16(eacol + h)],
                                       eas[h])
            pltpu.async_copy(msgv[b], acc.at[dst_v.at[j]], ssM[b], add=True)

            @pl.when(j + 3 < nb)
            def _():
                pltpu.async_copy(Hs.at[tix].at[src_v.at[j + 3]], rows[b],
                                 sgH[b])
                pltpu.async_copy(Ad.at[tix].at[dst_v.at[j + 3]], adv[b],
                                 sgA[b])

        @pl.loop(0, 3 * (nb // 3), step=3)
        def _blk(j0):
            for b in range(3):
                do_block(j0 + b, b)

        for j in range(3 * (nb // 3), nb):
            do_block(j, j % 3)

        for b in range(3):
            pltpu.make_async_copy(msgv[b], acc.at[dst_v.at[0]],
                                  ssM[b]).wait()

        plsc.subcore_barrier()

        for t in range(TCH):
            base = (sid * TCH + t) * RCH
            sl = pl.ds(base, RCH)

            @pl.when((cid == 0) & (base < N))
            def _():
                pltpu.async_copy(acc.at[sl], num0.at[sl], sems[0])

            @pl.when((cid == 1) & (base < N))
            def _():
                pltpu.async_copy(acc.at[sl], num1.at[sl], sems[0])
        for t in range(TCH):
            base = (sid * TCH + t) * RCH
            sl = pl.ds(base, RCH)

            @pl.when((cid == 0) & (base < N))
            def _():
                pltpu.make_async_copy(acc.at[sl], num0.at[sl],
                                      sems[0]).wait()

            @pl.when((cid == 1) & (base < N))
            def _():
                pltpu.make_async_copy(acc.at[sl], num1.at[sl],
                                      sems[0]).wait()

    return sc_pass


_sc_pass1 = _make_sc_pass(ROWW1, H1 // 2, C1, 64, 64, 72, 2, NB1)
_sc_pass2 = _make_sc_pass(ROWW2, 1, C2, 40, 40, 48, 1, NB2)


_RB = 1000


def _tc1_body(x_ref, w_ref, ms_ref, md_ref, hs_ref, ad_ref):
    h = jnp.dot(x_ref[...], w_ref[...], preferred_element_type=jnp.float32)
    a_s = jnp.dot(h, ms_ref[...], preferred_element_type=jnp.float32)
    a_d = jnp.dot(h, md_ref[...], preferred_element_type=jnp.float32)
    z12 = jnp.zeros((h.shape[0], 4), jnp.float32)
    z4 = jnp.zeros((h.shape[0], 4), jnp.float32)
    hs_ref[0] = jnp.concatenate([h[:, :64], a_s[:, :4], z12], axis=1)
    hs_ref[1] = jnp.concatenate([h[:, 64:], a_s[:, 4:], z12], axis=1)
    ad_ref[0] = jnp.concatenate([a_d[:, :4], z4], axis=1)
    ad_ref[1] = jnp.concatenate([a_d[:, 4:], z4], axis=1)


def _tc1(x, W1, Msrc, Mdst):
    return pl.pallas_call(
        _tc1_body,
        grid=(N // _RB,),
        in_specs=[
            pl.BlockSpec((_RB, D_IN), lambda i: (i, 0)),
            pl.BlockSpec((D_IN, D_IN), lambda i: (0, 0)),
            pl.BlockSpec((D_IN, H1), lambda i: (0, 0)),
            pl.BlockSpec((D_IN, H1), lambda i: (0, 0)),
        ],
        out_specs=[
            pl.BlockSpec((2, _RB, ROWW1), lambda i: (0, i, 0)),
            pl.BlockSpec((2, _RB, ADW), lambda i: (0, i, 0)),
        ],
        out_shape=[
            jax.ShapeDtypeStruct((2, N, ROWW1), jnp.float32),
            jax.ShapeDtypeStruct((2, N, ADW), jnp.float32),
        ],
    )(x, W1, Msrc, Mdst)


def _tc2_body(n0_ref, n1_ref, ex_ref, b1_ref, w2_ref,
              as2_ref, ad2_ref, hs_ref, ad_ref):
    num = jnp.concatenate([n0_ref[:, :64], n1_ref[:, :64]], axis=1)
    den = jnp.concatenate([n0_ref[:, 64:68], n1_ref[:, 64:68]], axis=1)
    den128 = jnp.dot(den, ex_ref[...], preferred_element_type=jnp.float32)
    out1 = num / (den128 + 1e-16) + b1_ref[...]
    y = jnp.where(out1 > 0.0, out1, jnp.exp(out1) - 1.0)
    h2 = jnp.dot(y, w2_ref[...], preferred_element_type=jnp.float32)
    a_s = jnp.dot(h2, as2_ref[...], preferred_element_type=jnp.float32)
    a_d = jnp.dot(h2, ad2_ref[...], preferred_element_type=jnp.float32)
    hs_ref[...] = jnp.concatenate([h2, a_s], axis=1)
    ad_ref[...] = a_d


def _tc2(n0, n1, Ex1, b1r, W2, as2m, ad2m):
    return pl.pallas_call(
        _tc2_body,
        grid=(N // _RB,),
        in_specs=[
            pl.BlockSpec((_RB, 72), lambda i: (i, 0)),
            pl.BlockSpec((_RB, 72), lambda i: (i, 0)),
            pl.BlockSpec((H1, 128), lambda i: (0, 0)),
            pl.BlockSpec((1, 128), lambda i: (0, 0)),
            pl.BlockSpec((128, C2), lambda i: (0, 0)),
            pl.BlockSpec((C2, 8), lambda i: (0, 0)),
            pl.BlockSpec((C2, ADW), lambda i: (0, 0)),
        ],
        out_specs=[
            pl.BlockSpec((_RB, ROWW2), lambda i: (i, 0)),
            pl.BlockSpec((_RB, ADW), lambda i: (i, 0)),
        ],
        out_shape=[
            jax.ShapeDtypeStruct((N, ROWW2), jnp.float32),
            jax.ShapeDtypeStruct((N, ADW), jnp.float32),
        ],
    )(n0, n1, Ex1, b1r, W2, as2m, ad2m)


def _tc3_body(n0_ref, n1_ref, ex_ref, b2_ref, o_ref):
    tot = n0_ref[...] + n1_ref[...]
    den40 = jnp.dot(tot[:, 40:48], ex_ref[...],
                    preferred_element_type=jnp.float32)
    o_ref[...] = tot[:, :C2] / (den40 + 1e-16) + b2_ref[...]


def _tc3(n0, n1, Ex2, b2r):
    return pl.pallas_call(
        _tc3_body,
        grid=(N // _RB,),
        in_specs=[
            pl.BlockSpec((_RB, ROWW2), lambda i: (i, 0)),
            pl.BlockSpec((_RB, ROWW2), lambda i: (i, 0)),
            pl.BlockSpec((ADW, C2), lambda i: (0, 0)),
            pl.BlockSpec((1, C2), lambda i: (0, 0)),
        ],
        out_specs=pl.BlockSpec((_RB, C2), lambda i: (i, 0)),
        out_shape=jax.ShapeDtypeStruct((N, C2), jnp.float32),
    )(n0, n1, Ex2, b2r)


_EX1 = np.zeros((H1, 128), np.float32)
for _h in range(H1):
    _EX1[_h, _h * C1:(_h + 1) * C1] = 1.0
_EX2 = np.zeros((ADW, C2), np.float32)
_EX2[0, :] = 1.0


def kernel(x, edge_index, W1, att_src1, att_dst1, b1, W2, att_src2,
           att_dst2, b2):
    srcI1 = edge_index[0].reshape(NS, NB1, B)
    dstI1 = edge_index[1].reshape(NS, NB1, B)
    srcI2 = edge_index[0].reshape(NW, NB2, B)
    dstI2 = edge_index[1].reshape(NW, NB2, B)

    a1s = att_src1.reshape(H1, C1)
    a1d = att_dst1.reshape(H1, C1)
    eye8 = jnp.eye(H1, dtype=jnp.float32)
    Msrc = (a1s[:, :, None] * eye8[:, None, :]).reshape(H1 * C1, H1)
    Mdst = (a1d[:, :, None] * eye8[:, None, :]).reshape(H1 * C1, H1)

    Hs1, Ad1 = _tc1(x, W1, Msrc, Mdst)
    n10, n11 = _sc_pass1(srcI1, dstI1, Hs1, Ad1)

    as2m = jnp.pad(att_src2.reshape(C2, 1), ((0, 0), (0, 7)))
    ad2m = jnp.pad(att_dst2.reshape(C2, 1), ((0, 0), (0, ADW - 1)))
    Hs2, Ad2 = _tc2(n10, n11, jnp.asarray(_EX1),
                    b1.reshape(1, 128), W2, as2m, ad2m)
    n20, n21 = _sc_pass2(srcI2, dstI2, Hs2[None], Ad2[None])

    return _tc3(n20, n21, jnp.asarray(_EX2), b2.reshape(1, C2))
